# Initial kernel scaffold; baseline (speedup 1.0000x reference)
import jax
import jax.numpy as jnp
from jax.experimental import pallas as pl


def kernel(x, gate_w, w_in, w_v, w_out):
    raise NotImplementedError("write your pallas kernel here")


# speedup vs baseline: 2.3436x; 2.3436x over previous
import functools

import jax
import jax.numpy as jnp
from jax.experimental import pallas as pl
from jax.experimental.pallas import tpu as pltpu

S = 2048
D_MODEL = 1024
E = 8
D_FF = 4096
F_BLK = 512
NF = D_FF // F_BLK


def _router_body(x_ref, gw_ref, probs_ref, comb_ref):
    x = x_ref[...]
    gw = gw_ref[...]
    logits = jax.lax.dot_general(
        x, gw, (((1,), (1,)), ((), ())), preferred_element_type=jnp.float32)
    m = jnp.max(logits, axis=-1, keepdims=True)
    ex = jnp.exp(logits - m)
    probs = ex / jnp.sum(ex, axis=-1, keepdims=True)
    probs_ref[...] = probs

    iota = jax.lax.broadcasted_iota(jnp.int32, probs.shape, 1)
    m1 = jnp.max(probs, axis=-1, keepdims=True)
    i1 = jnp.min(jnp.where(probs == m1, iota, E), axis=-1, keepdims=True)
    mask1 = iota == i1
    probs_lo = jnp.where(mask1, -1.0, probs)
    m2 = jnp.max(probs_lo, axis=-1, keepdims=True)
    i2 = jnp.min(jnp.where(probs_lo == m2, iota, E), axis=-1, keepdims=True)
    mask2 = iota == i2
    s = m1 + m2
    comb_ref[...] = jnp.where(mask1, m1 / s, 0.0) + jnp.where(mask2, m2 / s, 0.0)


def _ffn_body(x_ref, comb_ref, wi_ref, wv_ref, wo_ref, out_ref):
    e = pl.program_id(0)
    f = pl.program_id(1)

    @pl.when(jnp.logical_and(e == 0, f == 0))
    def _():
        out_ref[...] = jnp.zeros_like(out_ref)

    x16 = x_ref[...].astype(jnp.bfloat16)
    wi = wi_ref[0].astype(jnp.bfloat16)
    wv = wv_ref[0].astype(jnp.bfloat16)
    wo = wo_ref[0].astype(jnp.bfloat16)

    a = jax.lax.dot_general(
        x16, wi, (((1,), (1,)), ((), ())), preferred_element_type=jnp.float32)
    v = jax.lax.dot_general(
        x16, wv, (((1,), (1,)), ((), ())), preferred_element_type=jnp.float32)
    g = 0.5 * a * (1.0 + jax.lax.erf(a * 0.7071067811865476))
    h = (g * v).astype(jnp.bfloat16)
    part = jax.lax.dot_general(
        h, wo, (((1,), (1,)), ((), ())), preferred_element_type=jnp.float32)

    iota_e = jax.lax.broadcasted_iota(jnp.int32, comb_ref.shape, 1)
    col = jnp.sum(jnp.where(iota_e == e, comb_ref[...], 0.0), axis=1,
                  keepdims=True)
    out_ref[...] += col * part


def kernel(x, gate_w, w_in, w_v, w_out):
    x2 = x.reshape(S, D_MODEL)

    probs, comb = pl.pallas_call(
        _router_body,
        out_shape=(
            jax.ShapeDtypeStruct((S, E), jnp.float32),
            jax.ShapeDtypeStruct((S, E), jnp.float32),
        ),
    )(x2, gate_w)

    out = pl.pallas_call(
        _ffn_body,
        grid=(E, NF),
        in_specs=[
            pl.BlockSpec((S, D_MODEL), lambda e, f: (0, 0)),
            pl.BlockSpec((S, E), lambda e, f: (0, 0)),
            pl.BlockSpec((1, F_BLK, D_MODEL), lambda e, f: (e, f, 0)),
            pl.BlockSpec((1, F_BLK, D_MODEL), lambda e, f: (e, f, 0)),
            pl.BlockSpec((1, D_MODEL, F_BLK), lambda e, f: (e, 0, f)),
        ],
        out_specs=pl.BlockSpec((S, D_MODEL), lambda e, f: (0, 0)),
        out_shape=jax.ShapeDtypeStruct((S, D_MODEL), jnp.float32),
    )(x2, comb, w_in, w_v, w_out)

    return out.reshape(1, S, D_MODEL), probs.reshape(1, S, E)
